# baseline (speedup 1.0000x reference)
import jax
import jax.numpy as jnp
from jax.experimental import pallas as pl
from jax.experimental.pallas import tpu as pltpu

_E_CHK = 256
_PAD = 128


def _fused_kernel(x_ref, w1_ref, b1_ref, mv_ref, wo_ref, g1_ref, be1_ref,
                  w2_ref, b2_ref, g2_ref, be2_ref, out_ref, acc_ref):
    j = pl.program_id(0)
    nj = pl.num_programs(0)

    part = jax.lax.dot_general(
        x_ref[...], w1_ref[...],
        dimension_numbers=(((1,), (1,)), ((), ())),
        preferred_element_type=jnp.float32)

    @pl.when(j == 0)
    def _():
        acc_ref[...] = part

    @pl.when(j > 0)
    def _():
        acc_ref[...] += part

    @pl.when(j == nj - 1)
    def _():
        dd = w1_ref.shape[0]
        oo = w2_ref.shape[0]
        kk = mv_ref.shape[0]
        hh = wo_ref.shape[1] // mv_ref.shape[1]

        vmean = jnp.sum(mv_ref[...], axis=0, keepdims=True) / kk
        c_hv = jnp.concatenate([vmean] * hh, axis=1)
        c_vec = jax.lax.dot_general(
            c_hv, wo_ref[...], dimension_numbers=(((1,), (1,)), ((), ())),
            preferred_element_type=jnp.float32)
        w2g = w2_ref[...] * g1_ref[...]
        s_row = jax.lax.dot_general(
            g1_ref[...], w2_ref[...],
            dimension_numbers=(((1,), (1,)), ((), ())),
            preferred_element_type=jnp.float32)
        t_row = jax.lax.dot_general(
            be1_ref[...], w2_ref[...],
            dimension_numbers=(((1,), (1,)), ((), ())),
            preferred_element_type=jnp.float32) + b2_ref[...]
        cp_row = jax.lax.dot_general(
            c_vec, w2g, dimension_numbers=(((1,), (1,)), ((), ())),
            preferred_element_type=jnp.float32)
        sum_c = jnp.sum(c_vec, axis=1, keepdims=True)
        sum_c2 = jnp.sum(c_vec * c_vec, axis=1, keepdims=True)
        waug = jnp.concatenate(
            [w2g, jnp.ones((1, dd), jnp.float32), c_vec,
             jnp.zeros((_PAD - oo - 2, dd), jnp.float32)], axis=0)

        h = jnp.maximum(acc_ref[...] + b1_ref[...], 0.0)
        hsq = jnp.sum(h * h, axis=1, keepdims=True)
        p = jax.lax.dot_general(
            h, waug, dimension_numbers=(((1,), (1,)), ((), ())),
            preferred_element_type=jnp.float32)

        mean1 = (p[:, oo:oo + 1] + sum_c) / dd
        e2 = (hsq + 2.0 * p[:, oo + 1:oo + 2] + sum_c2) / dd
        var1 = e2 - mean1 * mean1
        rstd1 = 1.0 / jnp.sqrt(var1 + 1e-5)
        x2 = (p[:, 0:oo] + cp_row - mean1 * s_row) * rstd1 + t_row

        mean2 = jnp.mean(x2, axis=1, keepdims=True)
        cen2 = x2 - mean2
        var2 = jnp.mean(cen2 * cen2, axis=1, keepdims=True)
        y = cen2 / jnp.sqrt(var2 + 1e-5) * g2_ref[...] + be2_ref[...]
        out_ref[...] = jax.nn.sigmoid(y)


def kernel(X, W1, b1, mem_keys, mem_values, Wq, Wo, ln1_g, ln1_b,
           W2, b2, ln2_g, ln2_b):
    del mem_keys, Wq
    B, D = X.shape
    O = W2.shape[0]
    grid = (D // _E_CHK,)

    def echunk(j):
        return (0, j)

    def whole(j):
        return (0, 0)

    return pl.pallas_call(
        _fused_kernel,
        grid=grid,
        in_specs=[
            pl.BlockSpec((B, _E_CHK), echunk),
            pl.BlockSpec((D, _E_CHK), echunk),
            pl.BlockSpec((1, D), whole),
            pl.BlockSpec(mem_values.shape, whole),
            pl.BlockSpec(Wo.shape, whole),
            pl.BlockSpec((1, D), whole),
            pl.BlockSpec((1, D), whole),
            pl.BlockSpec(W2.shape, whole),
            pl.BlockSpec((1, O), whole),
            pl.BlockSpec((1, O), whole),
            pl.BlockSpec((1, O), whole),
        ],
        out_specs=pl.BlockSpec((B, O), whole),
        out_shape=jax.ShapeDtypeStruct((B, O), jnp.float32),
        scratch_shapes=[pltpu.VMEM((B, D), jnp.float32)],
    )(X, W1, b1.reshape(1, D), mem_values, Wo,
      ln1_g.reshape(1, D), ln1_b.reshape(1, D), W2,
      b2.reshape(1, O), ln2_g.reshape(1, O), ln2_b.reshape(1, O))


# speedup vs baseline: 1.1103x; 1.1103x over previous
import jax
import jax.numpy as jnp
from jax.experimental import pallas as pl

_PAD = 128


def _fused_kernel(x_ref, w1_ref, b1_ref, mv_ref, wo_ref, g1_ref, be1_ref,
                  w2_ref, b2_ref, g2_ref, be2_ref, out_ref):
    dd = w1_ref.shape[0]
    oo = w2_ref.shape[0]
    kk = mv_ref.shape[0]
    hh = wo_ref.shape[1] // mv_ref.shape[1]

    vmean = jnp.sum(mv_ref[...], axis=0, keepdims=True) / kk
    c_hv = jnp.concatenate([vmean] * hh, axis=1)
    c_vec = jax.lax.dot_general(
        c_hv, wo_ref[...], dimension_numbers=(((1,), (1,)), ((), ())),
        preferred_element_type=jnp.float32)
    w2g = w2_ref[...] * g1_ref[...]
    s_row = jax.lax.dot_general(
        g1_ref[...], w2_ref[...], dimension_numbers=(((1,), (1,)), ((), ())),
        preferred_element_type=jnp.float32)
    t_row = jax.lax.dot_general(
        be1_ref[...], w2_ref[...], dimension_numbers=(((1,), (1,)), ((), ())),
        preferred_element_type=jnp.float32) + b2_ref[...]
    cp_row = jax.lax.dot_general(
        c_vec, w2g, dimension_numbers=(((1,), (1,)), ((), ())),
        preferred_element_type=jnp.float32)
    sum_c = jnp.sum(c_vec, axis=1, keepdims=True)
    sum_c2 = jnp.sum(c_vec * c_vec, axis=1, keepdims=True)
    waug = jnp.concatenate(
        [w2g, jnp.ones((1, dd), jnp.float32), c_vec,
         jnp.zeros((_PAD - oo - 2, dd), jnp.float32)], axis=0)

    h = jax.lax.dot_general(
        x_ref[...], w1_ref[...],
        dimension_numbers=(((1,), (1,)), ((), ())),
        preferred_element_type=jnp.float32)
    h = jnp.maximum(h + b1_ref[...], 0.0)
    ones_d = jnp.ones((_PAD, dd), jnp.float32)
    hsq = jax.lax.dot_general(
        h * h, ones_d, dimension_numbers=(((1,), (1,)), ((), ())),
        preferred_element_type=jnp.float32)[:, 0:1]
    p = jax.lax.dot_general(
        h, waug, dimension_numbers=(((1,), (1,)), ((), ())),
        preferred_element_type=jnp.float32)

    mean1 = (p[:, oo:oo + 1] + sum_c) / dd
    e2 = (hsq + 2.0 * p[:, oo + 1:oo + 2] + sum_c2) / dd
    var1 = e2 - mean1 * mean1
    rstd1 = 1.0 / jnp.sqrt(var1 + 1e-5)
    x2 = (p[:, 0:oo] + cp_row - mean1 * s_row) * rstd1 + t_row

    ones_o = jnp.ones((_PAD, oo), jnp.float32)
    s1 = jax.lax.dot_general(
        x2, ones_o, dimension_numbers=(((1,), (1,)), ((), ())),
        preferred_element_type=jnp.float32)[:, 0:1]
    s2 = jax.lax.dot_general(
        x2 * x2, ones_o, dimension_numbers=(((1,), (1,)), ((), ())),
        preferred_element_type=jnp.float32)[:, 0:1]
    mean2 = s1 / oo
    var2 = s2 / oo - mean2 * mean2
    y = (x2 - mean2) / jnp.sqrt(var2 + 1e-5) * g2_ref[...] + be2_ref[...]
    out_ref[...] = jax.nn.sigmoid(y)


def kernel(X, W1, b1, mem_keys, mem_values, Wq, Wo, ln1_g, ln1_b,
           W2, b2, ln2_g, ln2_b):
    del mem_keys, Wq
    B, D = X.shape
    O = W2.shape[0]

    def whole(j):
        return (0, 0)

    return pl.pallas_call(
        _fused_kernel,
        grid=(1,),
        in_specs=[
            pl.BlockSpec((B, D), whole),
            pl.BlockSpec((D, D), whole),
            pl.BlockSpec((1, D), whole),
            pl.BlockSpec(mem_values.shape, whole),
            pl.BlockSpec(Wo.shape, whole),
            pl.BlockSpec((1, D), whole),
            pl.BlockSpec((1, D), whole),
            pl.BlockSpec(W2.shape, whole),
            pl.BlockSpec((1, O), whole),
            pl.BlockSpec((1, O), whole),
            pl.BlockSpec((1, O), whole),
        ],
        out_specs=pl.BlockSpec((B, O), whole),
        out_shape=jax.ShapeDtypeStruct((B, O), jnp.float32),
    )(X, W1, b1.reshape(1, D), mem_values, Wo,
      ln1_g.reshape(1, D), ln1_b.reshape(1, D), W2,
      b2.reshape(1, O), ln2_g.reshape(1, O), ln2_b.reshape(1, O))
